# baseline (speedup 1.0000x reference)
import numpy as np
import jax
import jax.numpy as jnp
from jax import lax
from jax.experimental import pallas as pl
from jax.experimental.pallas import tpu as pltpu
from jax.experimental.pallas import tpu_sc as plsc

_N = 512
_B = 256
_NOUT = _N * (_N - 1) // 2
_NCORE = 2
_NSUB = 16
_CH = _NOUT // _NSUB
_GR = 16
_NROW = 2 * _GR
_NSLOT = 4
_TT = _B // _NCORE // _NSLOT

_LA = [256 * s + 120 for s in range(_NSUB)]
_OFFA = [(_GR * s) * (_GR * s - 1) // 2 for s in range(_NSUB)]
_OFFB = [(_GR * (31 - s)) * (_GR * (31 - s) - 1) // 2 for s in range(_NSUB)]


def _build_tables():
    lidx = np.zeros((_NSUB, _CH), np.int32)
    for s in range(_NSUB):
        glist = list(range(_GR * s, _GR * s + _GR)) + \
                list(range(_GR * (31 - s), _GR * (31 - s) + _GR))
        pieces = [np.arange(r, dtype=np.int32) + _N * q
                  for q, r in enumerate(glist)]
        li = np.concatenate(pieces)
        assert li.size == _CH
        lidx[s] = li
    return lidx.reshape(-1)


_LIDX_NP = _build_tables()


def _tril_body(xt, lidx, out, libuf, sts, obs, gsems, osems):
    sub = lax.axis_index("s")
    half = lax.axis_index("c")
    lbase = pl.multiple_of(sub * _CH, 8)
    pltpu.sync_copy(lidx.at[pl.ds(lbase, _CH)], libuf)
    rowa = _GR * sub
    rowb = _GR * 31 - _GR * sub

    def stage(t, st, gsem):
        b = half + _NCORE * t
        basea = pl.multiple_of(b * _N + rowa, 8)
        baseb = pl.multiple_of(b * _N + rowb, 8)
        ca = pltpu.make_async_copy(xt.at[pl.ds(basea, _GR), :],
                                   st.at[pl.ds(0, _GR), :], gsem)
        cb = pltpu.make_async_copy(xt.at[pl.ds(baseb, _GR), :],
                                   st.at[pl.ds(_GR, _GR), :], gsem)
        return ca, cb

    for i in range(_NSLOT):
        for cp in stage(i, sts[i], gsems[i]):
            cp.start()

    def emit_out(ob, b, osem):
        def branch(p):
            def go():
                la, offa, offb = _LA[p], _OFFA[p], _OFFB[p]
                basea = pl.multiple_of(b * _NOUT + offa, 8)
                baseb = pl.multiple_of(b * _NOUT + offb, 8)
                pltpu.async_copy(ob.at[pl.ds(0, la)],
                                 out.at[pl.ds(basea, la)], osem)
                pltpu.async_copy(ob.at[pl.ds(la, _CH - la)],
                                 out.at[pl.ds(baseb, _CH - la)], osem)
            return go
        lax.switch(sub, [branch(p) for p in range(_NSUB)])

    def slot(tt, i, st, ob, gsem, osem):
        t = _NSLOT * tt + i
        b = half + _NCORE * t
        for cp in stage(t, st, gsem):
            cp.wait()

        @pl.when(tt > 0)
        def _():
            pltpu.make_async_copy(ob, out.at[pl.ds(0, _CH)], osem).wait()

        @plsc.parallel_loop(0, _CH, step=16, unroll=8)
        def _gloop(idx):
            iv = libuf[pl.ds(idx, 16)]
            row = lax.shift_right_logical(iv, 9)
            col = lax.bitwise_and(iv, _N - 1)
            ob[pl.ds(idx, 16)] = plsc.load_gather(st, [row, col])

        emit_out(ob, b, osem)

        @pl.when(tt < _TT - 1)
        def _():
            for cp in stage(t + _NSLOT, st, gsem):
                cp.start()

    def body(tt, carry):
        for i in range(_NSLOT):
            slot(tt, i, sts[i], obs[i], gsems[i], osems[i])
        return carry

    lax.fori_loop(0, _TT, body, 0)

    for i in range(_NSLOT):
        pltpu.make_async_copy(obs[i], out.at[pl.ds(0, _CH)], osems[i]).wait()


@jax.jit
def _tril_gather(xt, lidx):
    info = plsc.get_sparse_core_info()
    assert info.num_cores == _NCORE and info.num_subcores == _NSUB
    mesh = plsc.VectorSubcoreMesh(core_axis_name="c", subcore_axis_name="s")
    return pl.kernel(
        _tril_body,
        mesh=mesh,
        out_type=jax.ShapeDtypeStruct((_B * _NOUT,), jnp.float32),
        scratch_types=[
            pltpu.VMEM((_CH,), jnp.int32),
            [pltpu.VMEM((_NROW, _N), jnp.float32)] * _NSLOT,
            [pltpu.VMEM((_CH,), jnp.float32)] * _NSLOT,
            [pltpu.SemaphoreType.DMA] * _NSLOT,
            [pltpu.SemaphoreType.DMA] * _NSLOT,
        ],
        compiler_params=pltpu.CompilerParams(needs_layout_passes=False),
    )(xt, lidx)


def kernel(X):
    xt = X.reshape(_B * _N, _N)
    flat = _tril_gather(xt, jnp.asarray(_LIDX_NP))
    return flat.reshape(_B, _NOUT)


# speedup vs baseline: 1.0904x; 1.0904x over previous
import numpy as np
import jax
import jax.numpy as jnp
from jax import lax
from jax.experimental import pallas as pl
from jax.experimental.pallas import tpu as pltpu
from jax.experimental.pallas import tpu_sc as plsc

_N = 512
_B = 256
_NOUT = _N * (_N - 1) // 2
_NBLK = _NOUT // 128
_NCORE = 2
_NSUB = 16
_GR = 16
_OBW = 8192

_OFF = [(_GR * g) * (_GR * g - 1) // 2 for g in range(33)]
_KS = [-(-_OFF[g] // 128) for g in range(33)]
_NK = [_KS[g + 1] - _KS[g] for g in range(32)]


def _build_tables():
    def rc_of_p(p):
        r = int((1 + np.sqrt(1 + 8 * p)) // 2)
        while r * (r - 1) // 2 > p:
            r -= 1
        while r * (r + 1) // 2 <= p:
            r += 1
        return r, p - r * (r - 1) // 2

    lidx = np.zeros((_NSUB, _OBW), np.int32)
    for s in range(_NSUB):
        ga, gb = s, 31 - s
        na = _NK[ga]
        for w in range(_OBW):
            if w < na * 128:
                p, g, segb = 128 * _KS[ga] + w, ga, 0
            elif w < (na + _NK[gb]) * 128:
                p, g, segb = 128 * _KS[gb] + (w - na * 128), gb, 1
            else:
                continue
            r, c = rc_of_p(p)
            if r < _GR * g + _GR:
                lidx[s, w] = (r - _GR * g + _GR * segb) * _N + c
            else:
                lidx[s, w] = 32 * _N + 128 * segb + c
    return lidx.reshape(-1)


_LIDX_NP = _build_tables()


def _tril_body(xt, lidx, out, libuf, st0, st1, ob, gs0, gs1, osem):
    sub = lax.axis_index("s")
    core = lax.axis_index("c")
    lbase = pl.multiple_of(sub * _OBW, 8)
    pltpu.sync_copy(lidx.at[pl.ds(lbase, _OBW)], libuf)
    rowa = _GR * sub
    rowb = _GR * 31 - _GR * sub

    def stage_copies(t, st, gsem):
        b = 8 * (2 * (t // 8) + core) + (t % 8)
        base = b * _N
        rta = pl.multiple_of(base + rowa + _GR, 8)
        rtb = pl.multiple_of(jnp.minimum(base + rowb + _GR, _B * _N - 8), 8)
        return (
            pltpu.make_async_copy(xt.at[pl.ds(pl.multiple_of(base + rowa, 8),
                                              _GR), :],
                                  st.at[pl.ds(0, _GR), :], gsem),
            pltpu.make_async_copy(xt.at[pl.ds(pl.multiple_of(base + rowb, 8),
                                              _GR), :],
                                  st.at[pl.ds(_GR, _GR), :], gsem),
            pltpu.make_async_copy(xt.at[pl.ds(rta, 1), pl.ds(0, 128)],
                                  st.at[pl.ds(32, 1), pl.ds(0, 128)], gsem),
            pltpu.make_async_copy(xt.at[pl.ds(rtb, 1), pl.ds(0, 128)],
                                  st.at[pl.ds(32, 1), pl.ds(128, 128)], gsem),
        )

    def out_switch(m, do_wait):
        def branch(p):
            def go():
                na, ka = _NK[p], _KS[p]
                nb, kb = _NK[31 - p], _KS[31 - p]
                r0 = pl.multiple_of(8 * m, 8)
                ca = pltpu.make_async_copy(
                    ob.at[:, pl.ds(0, na * 128)],
                    out.at[pl.ds(r0, 8), pl.ds(128 * ka, na * 128)], osem)
                cb = pltpu.make_async_copy(
                    ob.at[:, pl.ds(na * 128, nb * 128)],
                    out.at[pl.ds(r0, 8), pl.ds(128 * kb, nb * 128)], osem)
                if do_wait:
                    ca.wait()
                    cb.wait()
                else:
                    ca.start()
                    cb.start()
            return go
        lax.switch(sub, [branch(p) for p in range(_NSUB)])

    for cp in stage_copies(0, st0, gs0):
        cp.start()

    def step(t, st_cur, gs_cur, st_nxt, gs_nxt):
        q = t % 8
        m = 2 * (t // 8) + core
        for cp in stage_copies(t, st_cur, gs_cur):
            cp.wait()

        @pl.when(t < _B // _NCORE - 1)
        def _():
            for cp in stage_copies(t + 1, st_nxt, gs_nxt):
                cp.start()

        @pl.when(jnp.logical_and(q == 0, t >= 8))
        def _():
            out_switch(m, do_wait=True)

        @plsc.parallel_loop(0, _OBW, step=16, unroll=8)
        def _gloop(i):
            iv = libuf[pl.ds(i, 16)]
            row = lax.shift_right_logical(iv, 9)
            col = lax.bitwise_and(iv, _N - 1)
            ob[q, pl.ds(i, 16)] = plsc.load_gather(st_cur, [row, col])

        @pl.when(q == 7)
        def _():
            out_switch(m, do_wait=False)

    def body(j, carry):
        step(2 * j, st0, gs0, st1, gs1)
        step(2 * j + 1, st1, gs1, st0, gs0)
        return carry

    lax.fori_loop(0, _B // _NCORE // 2, body, 0)
    out_switch(0, do_wait=True)


@jax.jit
def _tril_gather(xt, lidx):
    info = plsc.get_sparse_core_info()
    assert info.num_cores == _NCORE and info.num_subcores == _NSUB
    mesh = plsc.VectorSubcoreMesh(core_axis_name="c", subcore_axis_name="s")
    return pl.kernel(
        _tril_body,
        mesh=mesh,
        out_type=jax.ShapeDtypeStruct((_B, _NOUT), jnp.float32),
        scratch_types=[
            pltpu.VMEM((_OBW,), jnp.int32),
            pltpu.VMEM((33, _N), jnp.float32),
            pltpu.VMEM((33, _N), jnp.float32),
            pltpu.VMEM((8, _OBW), jnp.float32),
            pltpu.SemaphoreType.DMA,
            pltpu.SemaphoreType.DMA,
            pltpu.SemaphoreType.DMA,
        ],
        compiler_params=pltpu.CompilerParams(needs_layout_passes=False),
    )(xt, lidx)


def kernel(X):
    xt = X.reshape(_B * _N, _N)
    return _tril_gather(xt, jnp.asarray(_LIDX_NP))


# speedup vs baseline: 1.2453x; 1.1420x over previous
import numpy as np
import jax
import jax.numpy as jnp
from jax import lax
from jax.experimental import pallas as pl
from jax.experimental.pallas import tpu as pltpu
from jax.experimental.pallas import tpu_sc as plsc

_N = 512
_B = 256
_NOUT = _N * (_N - 1) // 2
_NBLK = _NOUT // 128
_NCORE = 2
_NSUB = 16
_GR = 16
_OBW = 8192

_OFF = [(_GR * g) * (_GR * g - 1) // 2 for g in range(33)]
_KS = [-(-_OFF[g] // 128) for g in range(33)]
_NK = [_KS[g + 1] - _KS[g] for g in range(32)]


def _build_tables():
    def rc_of_p(p):
        r = int((1 + np.sqrt(1 + 8 * p)) // 2)
        while r * (r - 1) // 2 > p:
            r -= 1
        while r * (r + 1) // 2 <= p:
            r += 1
        return r, p - r * (r - 1) // 2

    lidx = np.zeros((_NSUB, _OBW), np.int32)
    for s in range(_NSUB):
        ga, gb = s, 31 - s
        na = _NK[ga]
        for w in range(_OBW):
            if w < na * 128:
                p, g, segb = 128 * _KS[ga] + w, ga, 0
            elif w < (na + _NK[gb]) * 128:
                p, g, segb = 128 * _KS[gb] + (w - na * 128), gb, 1
            else:
                continue
            r, c = rc_of_p(p)
            if r < _GR * g + _GR:
                lidx[s, w] = (r - _GR * g + _GR * segb) * _N + c
            else:
                lidx[s, w] = 32 * _N + 128 * segb + c
    return lidx.reshape(-1)


_LIDX_NP = _build_tables()


def _tril_body(xt, lidx, out, libuf, st0, st1, ob, gs0, gs1, osem):
    sub = lax.axis_index("s")
    core = lax.axis_index("c")
    lbase = pl.multiple_of(sub * _OBW, 8)
    pltpu.sync_copy(lidx.at[pl.ds(lbase, _OBW)], libuf)
    rowa = _GR * sub
    rowb = _GR * 31 - _GR * sub

    def stage_copies(t, st, gsem, wa, wb):
        b = 8 * (2 * (t // 8) + core) + (t % 8)
        base = b * _N
        rta = pl.multiple_of(base + rowa + _GR, 8)
        rtb = pl.multiple_of(jnp.minimum(base + rowb + _GR, _B * _N - 8), 8)
        return (
            pltpu.make_async_copy(xt.at[pl.ds(pl.multiple_of(base + rowa, 8),
                                              _GR), pl.ds(0, wa)],
                                  st.at[pl.ds(0, _GR), pl.ds(0, wa)], gsem),
            pltpu.make_async_copy(xt.at[pl.ds(pl.multiple_of(base + rowb, 8),
                                              _GR), pl.ds(0, wb)],
                                  st.at[pl.ds(_GR, _GR), pl.ds(0, wb)], gsem),
            pltpu.make_async_copy(xt.at[pl.ds(rta, 1), pl.ds(0, 128)],
                                  st.at[pl.ds(32, 1), pl.ds(0, 128)], gsem),
            pltpu.make_async_copy(xt.at[pl.ds(rtb, 1), pl.ds(0, 128)],
                                  st.at[pl.ds(32, 1), pl.ds(128, 128)], gsem),
        )

    def stage_all(fn_name, t, st, gsem):
        @pl.when(sub < 8)
        def _():
            for cp in stage_copies(t, st, gsem, 128, 512):
                getattr(cp, fn_name)()

        @pl.when(sub >= 8)
        def _():
            for cp in stage_copies(t, st, gsem, 256, 384):
                getattr(cp, fn_name)()

    def out_switch(m, do_wait):
        def branch(p):
            def go():
                na, ka = _NK[p], _KS[p]
                nb, kb = _NK[31 - p], _KS[31 - p]
                r0 = pl.multiple_of(8 * m, 8)
                ca = pltpu.make_async_copy(
                    ob.at[:, pl.ds(0, na * 128)],
                    out.at[pl.ds(r0, 8), pl.ds(128 * ka, na * 128)], osem)
                cb = pltpu.make_async_copy(
                    ob.at[:, pl.ds(na * 128, nb * 128)],
                    out.at[pl.ds(r0, 8), pl.ds(128 * kb, nb * 128)], osem)
                if do_wait:
                    ca.wait()
                    cb.wait()
                else:
                    ca.start()
                    cb.start()
            return go
        lax.switch(sub, [branch(p) for p in range(_NSUB)])

    stage_all("start", 0, st0, gs0)

    def step(t, st_cur, gs_cur, st_nxt, gs_nxt):
        q = t % 8
        m = 2 * (t // 8) + core
        stage_all("wait", t, st_cur, gs_cur)

        @pl.when(t < _B // _NCORE - 1)
        def _():
            stage_all("start", t + 1, st_nxt, gs_nxt)

        @pl.when(jnp.logical_and(q == 0, t >= 8))
        def _():
            out_switch(m, do_wait=True)

        @plsc.parallel_loop(0, _OBW, step=16, unroll=8)
        def _gloop(i):
            iv = libuf[pl.ds(i, 16)]
            row = lax.shift_right_logical(iv, 9)
            col = lax.bitwise_and(iv, _N - 1)
            ob[q, pl.ds(i, 16)] = plsc.load_gather(st_cur, [row, col])

        @pl.when(q == 7)
        def _():
            out_switch(m, do_wait=False)

    def body(j, carry):
        step(2 * j, st0, gs0, st1, gs1)
        step(2 * j + 1, st1, gs1, st0, gs0)
        return carry

    lax.fori_loop(0, _B // _NCORE // 2, body, 0)
    out_switch(0, do_wait=True)


@jax.jit
def _tril_gather(xt, lidx):
    info = plsc.get_sparse_core_info()
    assert info.num_cores == _NCORE and info.num_subcores == _NSUB
    mesh = plsc.VectorSubcoreMesh(core_axis_name="c", subcore_axis_name="s")
    return pl.kernel(
        _tril_body,
        mesh=mesh,
        out_type=jax.ShapeDtypeStruct((_B, _NOUT), jnp.float32),
        scratch_types=[
            pltpu.VMEM((_OBW,), jnp.int32),
            pltpu.VMEM((33, _N), jnp.float32),
            pltpu.VMEM((33, _N), jnp.float32),
            pltpu.VMEM((8, _OBW), jnp.float32),
            pltpu.SemaphoreType.DMA,
            pltpu.SemaphoreType.DMA,
            pltpu.SemaphoreType.DMA,
        ],
        compiler_params=pltpu.CompilerParams(needs_layout_passes=False),
    )(xt, lidx)


def kernel(X):
    xt = X.reshape(_B * _N, _N)
    return _tril_gather(xt, jnp.asarray(_LIDX_NP))


# speedup vs baseline: 1.3507x; 1.0847x over previous
import numpy as np
import jax
import jax.numpy as jnp
from jax import lax
from jax.experimental import pallas as pl
from jax.experimental.pallas import tpu as pltpu
from jax.experimental.pallas import tpu_sc as plsc

_N = 512
_B = 256
_NOUT = _N * (_N - 1) // 2
_NBLK = _NOUT // 128
_NCORE = 2
_NSUB = 16
_GR = 16
_OBW = 8192

_OFF = [(_GR * g) * (_GR * g - 1) // 2 for g in range(33)]
_KS = [-(-_OFF[g] // 128) for g in range(33)]
_NK = [_KS[g + 1] - _KS[g] for g in range(32)]


def _build_tables():
    def rc_of_p(p):
        r = int((1 + np.sqrt(1 + 8 * p)) // 2)
        while r * (r - 1) // 2 > p:
            r -= 1
        while r * (r + 1) // 2 <= p:
            r += 1
        return r, p - r * (r - 1) // 2

    lidx = np.zeros((_NSUB, _OBW), np.int32)
    for s in range(_NSUB):
        ga, gb = s, 31 - s
        na = _NK[ga]
        for w in range(_OBW):
            if w < na * 128:
                p, g, segb = 128 * _KS[ga] + w, ga, 0
            elif w < (na + _NK[gb]) * 128:
                p, g, segb = 128 * _KS[gb] + (w - na * 128), gb, 1
            else:
                continue
            r, c = rc_of_p(p)
            if r < _GR * g + _GR:
                lidx[s, w] = (r - _GR * g + _GR * segb) * _N + c
            else:
                lidx[s, w] = 32 * _N + 128 * segb + c
    return lidx.reshape(-1)


_LIDX_NP = _build_tables()


def _tril_body(xt, lidx, out, libuf, st0, st1, ob, gs0, gs1, osem):
    sub = lax.axis_index("s")
    core = lax.axis_index("c")
    lbase = pl.multiple_of(sub * _OBW, 8)
    pltpu.sync_copy(lidx.at[pl.ds(lbase, _OBW)], libuf)
    rowa = _GR * sub
    rowb = _GR * 31 - _GR * sub

    def stage_copies(t, st, gsem, wa, wb):
        b = 8 * (2 * (t // 8) + core) + (t % 8)
        base = b * _N
        rta = pl.multiple_of(base + rowa + _GR, 8)
        rtb = pl.multiple_of(jnp.minimum(base + rowb + _GR, _B * _N - 8), 8)
        return (
            pltpu.make_async_copy(xt.at[pl.ds(pl.multiple_of(base + rowa, 8),
                                              _GR), pl.ds(0, wa)],
                                  st.at[pl.ds(0, _GR), pl.ds(0, wa)], gsem),
            pltpu.make_async_copy(xt.at[pl.ds(pl.multiple_of(base + rowb, 8),
                                              _GR), pl.ds(0, wb)],
                                  st.at[pl.ds(_GR, _GR), pl.ds(0, wb)], gsem),
            pltpu.make_async_copy(xt.at[pl.ds(rta, 1), pl.ds(0, 128)],
                                  st.at[pl.ds(32, 1), pl.ds(0, 128)], gsem),
            pltpu.make_async_copy(xt.at[pl.ds(rtb, 1), pl.ds(0, 128)],
                                  st.at[pl.ds(32, 1), pl.ds(128, 128)], gsem),
        )

    def stage_all(fn_name, t, st, gsem):
        @pl.when(sub < 8)
        def _():
            for cp in stage_copies(t, st, gsem, 128, 512):
                getattr(cp, fn_name)()

        @pl.when(sub >= 8)
        def _():
            for cp in stage_copies(t, st, gsem, 256, 384):
                getattr(cp, fn_name)()

    def out_switch(m, do_wait):
        def branch(p):
            def go():
                na, ka = _NK[p], _KS[p]
                nb, kb = _NK[31 - p], _KS[31 - p]
                r0 = pl.multiple_of(8 * m, 8)
                ca = pltpu.make_async_copy(
                    ob.at[:, pl.ds(0, na * 128)],
                    out.at[pl.ds(r0, 8), pl.ds(128 * ka, na * 128)], osem)
                cb = pltpu.make_async_copy(
                    ob.at[:, pl.ds(na * 128, nb * 128)],
                    out.at[pl.ds(r0, 8), pl.ds(128 * kb, nb * 128)], osem)
                if do_wait:
                    ca.wait()
                    cb.wait()
                else:
                    ca.start()
                    cb.start()
            return go
        lax.switch(sub, [branch(p) for p in range(_NSUB)])

    stage_all("start", 0, st0, gs0)

    def step(t, st_cur, gs_cur, st_nxt, gs_nxt):
        q = t % 8
        m = 2 * (t // 8) + core

        @pl.when(t < _B // _NCORE - 1)
        def _():
            stage_all("start", t + 1, st_nxt, gs_nxt)

        stage_all("wait", t, st_cur, gs_cur)

        @pl.when(jnp.logical_and(q == 0, t >= 8))
        def _():
            out_switch(m, do_wait=True)

        @plsc.parallel_loop(0, _OBW, step=16, unroll=16)
        def _gloop(i):
            iv = libuf[pl.ds(i, 16)]
            row = lax.shift_right_logical(iv, 9)
            col = lax.bitwise_and(iv, _N - 1)
            ob[q, pl.ds(i, 16)] = plsc.load_gather(st_cur, [row, col])

        @pl.when(q == 7)
        def _():
            out_switch(m, do_wait=False)

    def body(j, carry):
        step(2 * j, st0, gs0, st1, gs1)
        step(2 * j + 1, st1, gs1, st0, gs0)
        return carry

    lax.fori_loop(0, _B // _NCORE // 2, body, 0)
    out_switch(0, do_wait=True)


@jax.jit
def _tril_gather(xt, lidx):
    info = plsc.get_sparse_core_info()
    assert info.num_cores == _NCORE and info.num_subcores == _NSUB
    mesh = plsc.VectorSubcoreMesh(core_axis_name="c", subcore_axis_name="s")
    return pl.kernel(
        _tril_body,
        mesh=mesh,
        out_type=jax.ShapeDtypeStruct((_B, _NOUT), jnp.float32),
        scratch_types=[
            pltpu.VMEM((_OBW,), jnp.int32),
            pltpu.VMEM((33, _N), jnp.float32),
            pltpu.VMEM((33, _N), jnp.float32),
            pltpu.VMEM((8, _OBW), jnp.float32),
            pltpu.SemaphoreType.DMA,
            pltpu.SemaphoreType.DMA,
            pltpu.SemaphoreType.DMA,
        ],
        compiler_params=pltpu.CompilerParams(needs_layout_passes=False),
    )(xt, lidx)


def kernel(X):
    xt = X.reshape(_B * _N, _N)
    return _tril_gather(xt, jnp.asarray(_LIDX_NP))
